# baseline (speedup 1.0000x reference)
import jax
import jax.numpy as jnp
from jax.experimental import pallas as pl
from jax.experimental.pallas import tpu as pltpu

N = 1024
D = 128
BM = 256


def _gin_kernel(a_ref, x_ref, w_ref, b_ref, o_ref):
    i = pl.program_id(0)
    z = jax.lax.dot_general(
        x_ref[...], w_ref[...], (((1,), (1,)), ((), ())),
        preferred_element_type=jnp.float32)
    aggr = jnp.dot(a_ref[...].astype(jnp.bfloat16), z.astype(jnp.bfloat16),
                   preferred_element_type=jnp.float32)
    zblk = jax.lax.dot_general(
        x_ref[pl.ds(i * BM, BM), :], w_ref[...], (((1,), (1,)), ((), ())),
        preferred_element_type=jnp.float32)
    o_ref[...] = jnp.maximum(aggr + zblk + b_ref[...], 0.0)


def kernel(A, X, W, b):
    return pl.pallas_call(
        _gin_kernel,
        grid=(N // BM,),
        in_specs=[
            pl.BlockSpec((BM, N), lambda i: (i, 0)),
            pl.BlockSpec((N, D), lambda i: (0, 0)),
            pl.BlockSpec((D, D), lambda i: (0, 0)),
            pl.BlockSpec((1, D), lambda i: (0, 0)),
        ],
        out_specs=pl.BlockSpec((BM, D), lambda i: (i, 0)),
        out_shape=jax.ShapeDtypeStruct((N, D), jnp.float32),
        compiler_params=pltpu.CompilerParams(
            dimension_semantics=("parallel",)),
    )(A, X, W, b.reshape(1, D))


# speedup vs baseline: 1.3849x; 1.3849x over previous
import jax
import jax.numpy as jnp
from jax.experimental import pallas as pl
from jax.experimental.pallas import tpu as pltpu

N = 1024
D = 128
BM = 512


def _gin_kernel(a_ref, x_ref, w_ref, b_ref, o_ref, z_ref, zb_ref):
    i = pl.program_id(0)

    @pl.when(i == 0)
    def _():
        z = jax.lax.dot_general(
            x_ref[...], w_ref[...], (((1,), (1,)), ((), ())),
            preferred_element_type=jnp.float32)
        z_ref[...] = z
        zb_ref[...] = z.astype(jnp.bfloat16)

    aggr = jnp.dot(a_ref[...].astype(jnp.bfloat16), zb_ref[...],
                   preferred_element_type=jnp.float32)
    o_ref[...] = jnp.maximum(
        aggr + z_ref[pl.ds(i * BM, BM), :] + b_ref[...], 0.0)


def kernel(A, X, W, b):
    return pl.pallas_call(
        _gin_kernel,
        grid=(N // BM,),
        in_specs=[
            pl.BlockSpec((BM, N), lambda i: (i, 0)),
            pl.BlockSpec((N, D), lambda i: (0, 0)),
            pl.BlockSpec((D, D), lambda i: (0, 0)),
            pl.BlockSpec((1, D), lambda i: (0, 0)),
        ],
        out_specs=pl.BlockSpec((BM, D), lambda i: (i, 0)),
        out_shape=jax.ShapeDtypeStruct((N, D), jnp.float32),
        scratch_shapes=[
            pltpu.VMEM((N, D), jnp.float32),
            pltpu.VMEM((N, D), jnp.bfloat16),
        ],
        compiler_params=pltpu.CompilerParams(
            disable_bounds_checks=True,
            disable_semaphore_checks=True,
            skip_device_barrier=True,
        ),
    )(A, X, W, b.reshape(1, D))


# speedup vs baseline: 1.4060x; 1.0152x over previous
import jax
import jax.numpy as jnp
from jax.experimental import pallas as pl
from jax.experimental.pallas import tpu as pltpu

N = 1024
D = 128
BM = 512


def _gin_kernel(a_ref, x_ref, w_ref, b_ref, o_ref, z_ref, zb_ref):
    i = pl.program_id(0)

    @pl.when(i == 0)
    def _():
        z = jax.lax.dot_general(
            x_ref[...], w_ref[...], (((1,), (1,)), ((), ())),
            preferred_element_type=jnp.float32)
        z_ref[...] = z
        zb_ref[...] = z.astype(jnp.bfloat16)

    aggr = jnp.dot(a_ref[...].astype(jnp.bfloat16), zb_ref[...],
                   preferred_element_type=jnp.float32)
    o_ref[...] = jnp.maximum(
        aggr + z_ref[pl.ds(i * BM, BM), :] + b_ref[...], 0.0)


def kernel(A, X, W, b):
    return pl.pallas_call(
        _gin_kernel,
        grid=(N // BM,),
        in_specs=[
            pl.BlockSpec((BM, N), lambda i: (i, 0)),
            pl.BlockSpec((N, D), lambda i: (0, 0)),
            pl.BlockSpec((D, D), lambda i: (0, 0)),
            pl.BlockSpec((1, D), lambda i: (0, 0)),
        ],
        out_specs=pl.BlockSpec((BM, D), lambda i: (i, 0)),
        out_shape=jax.ShapeDtypeStruct((N, D), jnp.float32),
        scratch_shapes=[
            pltpu.VMEM((N, D), jnp.float32),
            pltpu.VMEM((N, D), jnp.bfloat16),
        ],
    )(A, X, W, b.reshape(1, D))
